# Initial kernel scaffold; baseline (speedup 1.0000x reference)
import functools

import jax
import jax.numpy as jnp
from jax import lax
from jax.experimental import pallas as pl
from jax.experimental.pallas import tpu as pltpu
from jax.experimental.pallas import tpu_sc as plsc

_N = 10000
_E = 320000
_D = 128
_G = 64
_NC = 2
_NS = 16
_TILES = _NC * _NS
_C = 125
_E_ROWS = _E // _C
_CHUNKS = _E // (_TILES * _C)
_ROWS_PER_TILE = _N // _NS
_RCHUNKS = _ROWS_PER_TILE // _C

_LANES = 16


def _zero_vmem(ref, nrows, ncols):
    zv = jnp.zeros((_LANES,), jnp.float32)

    def _row(i, _):
        def _col(k, _):
            ref[i, pl.ds(k * _LANES, _LANES)] = zv
            return 0
        return lax.fori_loop(0, ncols // _LANES, _col, 0)

    lax.fori_loop(0, nrows, _row, 0)


def _sc_degree_body(dst_hbm, out_hbm, accum, dstv, buf):
    cid = lax.axis_index("c")
    sid = lax.axis_index("s")
    t = cid * _NS + sid

    pltpu.sync_copy(dst_hbm.at[pl.ds(t * _CHUNKS, _CHUNKS)], dstv)

    _zero_vmem(buf, _C, _LANES)

    def _zacc(k, _):
        pltpu.sync_copy(buf, accum.at[pl.ds(sid * _ROWS_PER_TILE + k * _C, _C)])
        return 0
    lax.fori_loop(0, _RCHUNKS, _zacc, 0)

    onev = jnp.where(lax.iota(jnp.int32, (_LANES,)) == 0, 1.0, 0.0)

    def _ones(i, _):
        buf[i, pl.ds(0, _LANES)] = onev
        return 0
    lax.fori_loop(0, _C, _ones, 0)

    plsc.subcore_barrier()

    def _chunk(j, _):
        pltpu.sync_copy(buf, accum.at[dstv.at[j]], add=True)
        return 0
    lax.fori_loop(0, _CHUNKS, _chunk, 0)

    plsc.subcore_barrier()

    def _wout(k, _):
        r0 = sid * _ROWS_PER_TILE + k * _C
        pltpu.sync_copy(accum.at[pl.ds(r0, _C)], buf)
        pltpu.sync_copy(buf, out_hbm.at[cid].at[pl.ds(r0, _C)])
        return 0
    lax.fori_loop(0, _RCHUNKS, _wout, 0)


_sc_degree = pl.kernel(
    _sc_degree_body,
    out_type=jax.ShapeDtypeStruct((_NC, _N, _LANES), jnp.float32),
    mesh=plsc.VectorSubcoreMesh(core_axis_name="c", subcore_axis_name="s"),
    scratch_types=[
        pltpu.VMEM_SHARED((_N, _LANES), jnp.float32),
        pltpu.VMEM((_CHUNKS, _C), jnp.int32),
        pltpu.VMEM((_C, _LANES), jnp.float32),
    ],
)


def _sc_scatter_body(z_hbm, src_hbm, dst_hbm, out_hbm,
                     accum, srcv, dstv, rows, sem):
    cid = lax.axis_index("c")
    sid = lax.axis_index("s")
    t = cid * _NS + sid

    pltpu.sync_copy(src_hbm.at[pl.ds(t * _CHUNKS, _CHUNKS)], srcv)
    pltpu.sync_copy(dst_hbm.at[pl.ds(t * _CHUNKS, _CHUNKS)], dstv)

    _zero_vmem(rows, _C, _D)

    def _zacc(k, _):
        pltpu.sync_copy(rows, accum.at[pl.ds(sid * _ROWS_PER_TILE + k * _C, _C)])
        return 0
    lax.fori_loop(0, _RCHUNKS, _zacc, 0)

    plsc.subcore_barrier()

    def _chunk(j, _):
        pltpu.async_copy(z_hbm.at[srcv.at[j]], rows, sem).wait()
        pltpu.sync_copy(rows, accum.at[dstv.at[j]], add=True)
        return 0
    lax.fori_loop(0, _CHUNKS, _chunk, 0)

    plsc.subcore_barrier()

    def _wout(k, _):
        r0 = sid * _ROWS_PER_TILE + k * _C
        pltpu.sync_copy(accum.at[pl.ds(r0, _C)], rows)
        pltpu.sync_copy(rows, out_hbm.at[cid].at[pl.ds(r0, _C)])
        return 0
    lax.fori_loop(0, _RCHUNKS, _wout, 0)


_sc_scatter = pl.kernel(
    _sc_scatter_body,
    out_type=jax.ShapeDtypeStruct((_NC, _N, _D), jnp.float32),
    mesh=plsc.VectorSubcoreMesh(core_axis_name="c", subcore_axis_name="s"),
    scratch_types=[
        pltpu.VMEM_SHARED((_N, _D), jnp.float32),
        pltpu.VMEM((_CHUNKS, _C), jnp.int32),
        pltpu.VMEM((_CHUNKS, _C), jnp.int32),
        pltpu.VMEM((_C, _D), jnp.float32),
        pltpu.SemaphoreType.DMA,
    ],
)


_BLK = 1000
_GRID = _N // _BLK


def _mm_body(x_ref, w_ref, o_ref):
    o_ref[...] = jnp.dot(x_ref[...], w_ref[...],
                         preferred_element_type=jnp.float32)


_mm = pl.pallas_call(
    _mm_body,
    grid=(_GRID,),
    in_specs=[
        pl.BlockSpec((_BLK, _D), lambda i: (i, 0)),
        pl.BlockSpec((_D, _D), lambda i: (0, 0)),
    ],
    out_specs=pl.BlockSpec((_BLK, _D), lambda i: (i, 0)),
    out_shape=jax.ShapeDtypeStruct((_N, _D), jnp.float32),
)


def _scale_body(xw_ref, ca_ref, cb_ref, z_ref, dinv_ref):
    deg = ca_ref[:, 0:1] + cb_ref[:, 0:1] + 1.0
    dinv = lax.rsqrt(deg)
    z_ref[...] = xw_ref[...] * dinv
    dinv_ref[...] = dinv


_scale = pl.pallas_call(
    _scale_body,
    grid=(_GRID,),
    in_specs=[
        pl.BlockSpec((_BLK, _D), lambda i: (i, 0)),
        pl.BlockSpec((_BLK, _LANES), lambda i: (i, 0)),
        pl.BlockSpec((_BLK, _LANES), lambda i: (i, 0)),
    ],
    out_specs=[
        pl.BlockSpec((_BLK, _D), lambda i: (i, 0)),
        pl.BlockSpec((_BLK, 1), lambda i: (i, 0)),
    ],
    out_shape=[
        jax.ShapeDtypeStruct((_N, _D), jnp.float32),
        jax.ShapeDtypeStruct((_N, 1), jnp.float32),
    ],
)


def _layer_body(sa_ref, sb_ref, z_ref, dinv_ref, b_ref, w_ref, zn_ref):
    dinv = dinv_ref[...]
    h = dinv * (sa_ref[...] + sb_ref[...] + z_ref[...]) + b_ref[...]
    h = jnp.maximum(h, 0.0)
    zn_ref[...] = jnp.dot(h, w_ref[...],
                          preferred_element_type=jnp.float32) * dinv


_layer = pl.pallas_call(
    _layer_body,
    grid=(_GRID,),
    in_specs=[
        pl.BlockSpec((_BLK, _D), lambda i: (i, 0)),
        pl.BlockSpec((_BLK, _D), lambda i: (i, 0)),
        pl.BlockSpec((_BLK, _D), lambda i: (i, 0)),
        pl.BlockSpec((_BLK, 1), lambda i: (i, 0)),
        pl.BlockSpec((1, _D), lambda i: (0, 0)),
        pl.BlockSpec((_D, _D), lambda i: (0, 0)),
    ],
    out_specs=pl.BlockSpec((_BLK, _D), lambda i: (i, 0)),
    out_shape=jax.ShapeDtypeStruct((_N, _D), jnp.float32),
)


def _head_body(sa_ref, sb_ref, z_ref, dinv_ref, b_ref, batch_ref,
               wf1_ref, bf1_ref, wf2_ref, bf2_ref, o_ref):
    h = dinv_ref[...] * (sa_ref[...] + sb_ref[...] + z_ref[...]) + b_ref[...]
    gids = lax.broadcasted_iota(jnp.int32, (_G, _N), 0)
    onehot = (batch_ref[...] == gids).astype(jnp.float32)
    sums = jnp.dot(onehot, h, preferred_element_type=jnp.float32)
    cnts = jnp.sum(onehot, axis=1, keepdims=True)
    g = sums / jnp.maximum(cnts, 1.0)
    a = jnp.maximum(
        jnp.dot(g, wf1_ref[...], preferred_element_type=jnp.float32)
        + bf1_ref[...], 0.0)
    o_ref[...] = (jnp.dot(a, wf2_ref[...], preferred_element_type=jnp.float32)
                  + bf2_ref[...])


_head = pl.pallas_call(
    _head_body,
    out_shape=jax.ShapeDtypeStruct((_G, 10), jnp.float32),
)


def kernel(x, edge_index, batch, W1, b1, W2, b2, W3, b3, Wf1, bf1, Wf2, bf2):
    src = edge_index[0].astype(jnp.int32).reshape(_E_ROWS, _C)
    dst = edge_index[1].astype(jnp.int32).reshape(_E_ROWS, _C)
    batch2d = batch.astype(jnp.int32).reshape(1, _N)

    cnts = _sc_degree(dst)
    xw1 = _mm(x, W1)
    z1, dinv = _scale(xw1, cnts[0], cnts[1])

    s1 = _sc_scatter(z1, src, dst)
    z2 = _layer(s1[0], s1[1], z1, dinv, b1.reshape(1, _D), W2)
    s2 = _sc_scatter(z2, src, dst)
    z3 = _layer(s2[0], s2[1], z2, dinv, b2.reshape(1, _D), W3)
    s3 = _sc_scatter(z3, src, dst)

    out = _head(s3[0], s3[1], z3, dinv, b3.reshape(1, _D), batch2d,
                Wf1, bf1.reshape(1, 64), Wf2, bf2.reshape(1, 10))
    return out


# speedup vs baseline: 17.5877x; 17.5877x over previous
import jax
import jax.numpy as jnp
from jax import lax
from jax.experimental import pallas as pl
from jax.experimental.pallas import tpu as pltpu
from jax.experimental.pallas import tpu_sc as plsc

_N = 10000
_NP = 10240
_E = 320000
_D = 128
_G = 64
_NC = 2
_NS = 16
_TILES = _NC * _NS
_C = 125
_E_ROWS = _E // _C
_CHUNKS = _E // (_TILES * _C)
_RPT = _NP // _NS
_RC = 128
_RCHUNKS = _RPT // _RC

_LANES = 16


def _zero_vmem(ref, nrows, ncols):
    zv = jnp.zeros((_LANES,), jnp.float32)

    def _row(i, _):
        def _col(k, _):
            ref[i, pl.ds(k * _LANES, _LANES)] = zv
            return 0
        return lax.fori_loop(0, ncols // _LANES, _col, 0)

    lax.fori_loop(0, nrows, _row, 0)


def _sc_degree_body(dst_hbm, out_hbm, accum, dstv, buf):
    cid = lax.axis_index("c")
    sid = lax.axis_index("s")
    t = cid * _NS + sid

    pltpu.sync_copy(dst_hbm.at[pl.ds(t * _CHUNKS, _CHUNKS)], dstv)

    _zero_vmem(buf, _RC, _D)

    def _zacc(k, _):
        pltpu.sync_copy(buf, accum.at[pl.ds(sid * _RPT + k * _RC, _RC)])
        return 0
    lax.fori_loop(0, _RCHUNKS, _zacc, 0)

    onev = jnp.where(lax.iota(jnp.int32, _LANES) == 0, 1.0, 0.0)

    def _ones(i, _):
        buf[i, pl.ds(0, _LANES)] = onev
        return 0
    lax.fori_loop(0, _C, _ones, 0)

    plsc.subcore_barrier()

    def _chunk(j, _):
        pltpu.sync_copy(buf.at[pl.ds(0, _C)], accum.at[dstv.at[j]], add=True)
        return 0
    lax.fori_loop(0, _CHUNKS, _chunk, 0)

    plsc.subcore_barrier()

    def _wout(k, _):
        r0 = sid * _RPT + k * _RC
        pltpu.sync_copy(accum.at[pl.ds(r0, _RC)], buf)
        pltpu.sync_copy(buf, out_hbm.at[cid].at[pl.ds(r0, _RC)])
        return 0
    lax.fori_loop(0, _RCHUNKS, _wout, 0)


_sc_degree = pl.kernel(
    _sc_degree_body,
    out_type=jax.ShapeDtypeStruct((_NC, _NP, _D), jnp.float32),
    mesh=plsc.VectorSubcoreMesh(core_axis_name="c", subcore_axis_name="s"),
    scratch_types=[
        pltpu.VMEM_SHARED((_NP, _D), jnp.float32),
        pltpu.VMEM((_CHUNKS, _C), jnp.int32),
        pltpu.VMEM((_RC, _D), jnp.float32),
    ],
)


def _sc_scatter_body(z_hbm, src_hbm, dst_hbm, out_hbm,
                     accum, srcv, dstv, rows, sem):
    cid = lax.axis_index("c")
    sid = lax.axis_index("s")
    t = cid * _NS + sid

    pltpu.sync_copy(src_hbm.at[pl.ds(t * _CHUNKS, _CHUNKS)], srcv)
    pltpu.sync_copy(dst_hbm.at[pl.ds(t * _CHUNKS, _CHUNKS)], dstv)

    _zero_vmem(rows, _RC, _D)

    def _zacc(k, _):
        pltpu.sync_copy(rows, accum.at[pl.ds(sid * _RPT + k * _RC, _RC)])
        return 0
    lax.fori_loop(0, _RCHUNKS, _zacc, 0)

    plsc.subcore_barrier()

    def _chunk(j, _):
        pltpu.async_copy(z_hbm.at[srcv.at[j]], rows.at[pl.ds(0, _C)],
                         sem).wait()
        pltpu.sync_copy(rows.at[pl.ds(0, _C)], accum.at[dstv.at[j]], add=True)
        return 0
    lax.fori_loop(0, _CHUNKS, _chunk, 0)

    plsc.subcore_barrier()

    def _wout(k, _):
        r0 = sid * _RPT + k * _RC
        pltpu.sync_copy(accum.at[pl.ds(r0, _RC)], rows)
        pltpu.sync_copy(rows, out_hbm.at[cid].at[pl.ds(r0, _RC)])
        return 0
    lax.fori_loop(0, _RCHUNKS, _wout, 0)


_sc_scatter = pl.kernel(
    _sc_scatter_body,
    out_type=jax.ShapeDtypeStruct((_NC, _NP, _D), jnp.float32),
    mesh=plsc.VectorSubcoreMesh(core_axis_name="c", subcore_axis_name="s"),
    scratch_types=[
        pltpu.VMEM_SHARED((_NP, _D), jnp.float32),
        pltpu.VMEM((_CHUNKS, _C), jnp.int32),
        pltpu.VMEM((_CHUNKS, _C), jnp.int32),
        pltpu.VMEM((_RC, _D), jnp.float32),
        pltpu.SemaphoreType.DMA,
    ],
)


_BLK = 1024
_GRID = _NP // _BLK


def _mm_body(x_ref, w_ref, o_ref):
    o_ref[...] = jnp.dot(x_ref[...], w_ref[...],
                         preferred_element_type=jnp.float32)


_mm = pl.pallas_call(
    _mm_body,
    grid=(_GRID,),
    in_specs=[
        pl.BlockSpec((_BLK, _D), lambda i: (i, 0)),
        pl.BlockSpec((_D, _D), lambda i: (0, 0)),
    ],
    out_specs=pl.BlockSpec((_BLK, _D), lambda i: (i, 0)),
    out_shape=jax.ShapeDtypeStruct((_NP, _D), jnp.float32),
)


def _scale_body(xw_ref, ca_ref, cb_ref, z_ref, dinv_ref):
    deg = ca_ref[:, 0:1] + cb_ref[:, 0:1] + 1.0
    dinv = lax.rsqrt(deg)
    z_ref[...] = xw_ref[...] * dinv
    dinv_ref[...] = dinv


_scale = pl.pallas_call(
    _scale_body,
    grid=(_GRID,),
    in_specs=[
        pl.BlockSpec((_BLK, _D), lambda i: (i, 0)),
        pl.BlockSpec((_BLK, _D), lambda i: (i, 0)),
        pl.BlockSpec((_BLK, _D), lambda i: (i, 0)),
    ],
    out_specs=[
        pl.BlockSpec((_BLK, _D), lambda i: (i, 0)),
        pl.BlockSpec((_BLK, 1), lambda i: (i, 0)),
    ],
    out_shape=[
        jax.ShapeDtypeStruct((_NP, _D), jnp.float32),
        jax.ShapeDtypeStruct((_NP, 1), jnp.float32),
    ],
)


def _layer_body(sa_ref, sb_ref, z_ref, dinv_ref, b_ref, w_ref, zn_ref):
    dinv = dinv_ref[...]
    h = dinv * (sa_ref[...] + sb_ref[...] + z_ref[...]) + b_ref[...]
    h = jnp.maximum(h, 0.0)
    zn_ref[...] = jnp.dot(h, w_ref[...],
                          preferred_element_type=jnp.float32) * dinv


_layer = pl.pallas_call(
    _layer_body,
    grid=(_GRID,),
    in_specs=[
        pl.BlockSpec((_BLK, _D), lambda i: (i, 0)),
        pl.BlockSpec((_BLK, _D), lambda i: (i, 0)),
        pl.BlockSpec((_BLK, _D), lambda i: (i, 0)),
        pl.BlockSpec((_BLK, 1), lambda i: (i, 0)),
        pl.BlockSpec((1, _D), lambda i: (0, 0)),
        pl.BlockSpec((_D, _D), lambda i: (0, 0)),
    ],
    out_specs=pl.BlockSpec((_BLK, _D), lambda i: (i, 0)),
    out_shape=jax.ShapeDtypeStruct((_NP, _D), jnp.float32),
)


def _head_body(sa_ref, sb_ref, z_ref, dinv_ref, b_ref, batch_ref,
               wf1_ref, bf1_ref, wf2_ref, bf2_ref, o_ref):
    h = dinv_ref[...] * (sa_ref[...] + sb_ref[...] + z_ref[...]) + b_ref[...]
    gids = lax.broadcasted_iota(jnp.int32, (_G, _NP), 0)
    onehot = (batch_ref[...] == gids).astype(jnp.float32)
    sums = jnp.dot(onehot, h, preferred_element_type=jnp.float32)
    cnts = jnp.sum(onehot, axis=1, keepdims=True)
    g = sums / jnp.maximum(cnts, 1.0)
    a = jnp.maximum(
        jnp.dot(g, wf1_ref[...], preferred_element_type=jnp.float32)
        + bf1_ref[...], 0.0)
    o_ref[...] = (jnp.dot(a, wf2_ref[...], preferred_element_type=jnp.float32)
                  + bf2_ref[...])


_head = pl.pallas_call(
    _head_body,
    out_shape=jax.ShapeDtypeStruct((_G, 10), jnp.float32),
)


def kernel(x, edge_index, batch, W1, b1, W2, b2, W3, b3, Wf1, bf1, Wf2, bf2):
    src = edge_index[0].astype(jnp.int32).reshape(_E_ROWS, _C)
    dst = edge_index[1].astype(jnp.int32).reshape(_E_ROWS, _C)
    x_pad = jnp.pad(x, ((0, _NP - _N), (0, 0)))
    batch2d = jnp.pad(batch.astype(jnp.int32), (0, _NP - _N),
                      constant_values=_G).reshape(1, _NP)

    cnts = _sc_degree(dst)
    xw1 = _mm(x_pad, W1)
    z1, dinv = _scale(xw1, cnts[0], cnts[1])

    s1 = _sc_scatter(z1, src, dst)
    z2 = _layer(s1[0], s1[1], z1, dinv, b1.reshape(1, _D), W2)
    s2 = _sc_scatter(z2, src, dst)
    z3 = _layer(s2[0], s2[1], z2, dinv, b2.reshape(1, _D), W3)
    s3 = _sc_scatter(z3, src, dst)

    out = _head(s3[0], s3[1], z3, dinv, b3.reshape(1, _D), batch2d,
                Wf1, bf1.reshape(1, 64), Wf2, bf2.reshape(1, 10))
    return out


# speedup vs baseline: 23.4516x; 1.3334x over previous
import jax
import jax.numpy as jnp
from jax import lax
from jax.experimental import pallas as pl
from jax.experimental.pallas import tpu as pltpu
from jax.experimental.pallas import tpu_sc as plsc

_N = 10000
_NP = 10240
_E = 320000
_D = 128
_G = 64
_NC = 2
_NS = 16
_TILES = _NC * _NS
_C = 125
_E_ROWS = _E // _C
_CHUNKS = _E // (_TILES * _C)
_RPT = _NP // _NS
_RC = 128
_RCHUNKS = _RPT // _RC

_LANES = 16


def _zero_vmem(ref, nrows, ncols):
    zv = jnp.zeros((_LANES,), jnp.float32)

    def _row(i, _):
        def _col(k, _):
            ref[i, pl.ds(k * _LANES, _LANES)] = zv
            return 0
        return lax.fori_loop(0, ncols // _LANES, _col, 0)

    lax.fori_loop(0, nrows, _row, 0)


def _sc_degree_body(dst_hbm, out_hbm, accum, dstv, buf):
    cid = lax.axis_index("c")
    sid = lax.axis_index("s")
    t = cid * _NS + sid

    pltpu.sync_copy(dst_hbm.at[pl.ds(t * _CHUNKS, _CHUNKS)], dstv)

    _zero_vmem(buf, _RC, _D)

    def _zacc(k, _):
        pltpu.sync_copy(buf, accum.at[pl.ds(sid * _RPT + k * _RC, _RC)])
        return 0
    lax.fori_loop(0, _RCHUNKS, _zacc, 0)

    onev = jnp.where(lax.iota(jnp.int32, _LANES) == 0, 1.0, 0.0)

    def _ones(i, _):
        buf[i, pl.ds(0, _LANES)] = onev
        return 0
    lax.fori_loop(0, _C, _ones, 0)

    plsc.subcore_barrier()

    def _chunk(j, _):
        pltpu.sync_copy(buf.at[pl.ds(0, _C)], accum.at[dstv.at[j]], add=True)
        return 0
    lax.fori_loop(0, _CHUNKS, _chunk, 0)

    plsc.subcore_barrier()

    def _wout(k, _):
        r0 = sid * _RPT + k * _RC
        pltpu.sync_copy(accum.at[pl.ds(r0, _RC)], buf)
        pltpu.sync_copy(buf, out_hbm.at[cid].at[pl.ds(r0, _RC)])
        return 0
    lax.fori_loop(0, _RCHUNKS, _wout, 0)


_sc_degree = pl.kernel(
    _sc_degree_body,
    out_type=jax.ShapeDtypeStruct((_NC, _NP, _D), jnp.float32),
    mesh=plsc.VectorSubcoreMesh(core_axis_name="c", subcore_axis_name="s"),
    scratch_types=[
        pltpu.VMEM_SHARED((_NP, _D), jnp.float32),
        pltpu.VMEM((_CHUNKS, _C), jnp.int32),
        pltpu.VMEM((_RC, _D), jnp.float32),
    ],
)


_QB = 5
_QC = _CHUNKS // _QB


def _sc_scatter_body(z_hbm, src_hbm, dst_hbm, out_hbm,
                     accum, srcv, dstv, bufa, bufb,
                     gsema, gsemb, ssema, ssemb):
    cid = lax.axis_index("c")
    sid = lax.axis_index("s")
    t = cid * _NS + sid

    _zero_vmem(bufa, _RC, _D)

    def _zacc(k, _):
        pltpu.sync_copy(bufa, accum.at[pl.ds(sid * _RPT + k * _RC, _RC)])
        return 0
    lax.fori_loop(0, _RCHUNKS, _zacc, 0)

    plsc.subcore_barrier()

    bufs = (bufa, bufb)
    gsems = (gsema, gsemb)
    ssems = (ssema, ssemb)

    def _qblock(q, _):
        base = t * _CHUNKS + q * _QC
        pltpu.sync_copy(src_hbm.at[pl.ds(base, _QC)], srcv)
        pltpu.sync_copy(dst_hbm.at[pl.ds(base, _QC)], dstv)

        gd = [None, None]
        sd = [None, None]
        gd[0] = pltpu.async_copy(z_hbm.at[srcv.at[0]],
                                 bufs[0].at[pl.ds(0, _C)], gsems[0])
        for j in range(_QC):
            p = j % 2
            o = (j + 1) % 2
            if j + 1 < _QC:
                if sd[o] is not None:
                    sd[o].wait()
                gd[o] = pltpu.async_copy(z_hbm.at[srcv.at[j + 1]],
                                         bufs[o].at[pl.ds(0, _C)], gsems[o])
            gd[p].wait()
            sd[p] = pltpu.async_copy(bufs[p].at[pl.ds(0, _C)],
                                     accum.at[dstv.at[j]], ssems[p], add=True)
        sd[0].wait()
        sd[1].wait()
        return 0
    lax.fori_loop(0, _QB, _qblock, 0)

    plsc.subcore_barrier()

    def _wout(k, _):
        r0 = sid * _RPT + k * _RC
        pltpu.sync_copy(accum.at[pl.ds(r0, _RC)], bufa)
        pltpu.sync_copy(bufa, out_hbm.at[cid].at[pl.ds(r0, _RC)])
        return 0
    lax.fori_loop(0, _RCHUNKS, _wout, 0)


_sc_scatter = pl.kernel(
    _sc_scatter_body,
    out_type=jax.ShapeDtypeStruct((_NC, _NP, _D), jnp.float32),
    mesh=plsc.VectorSubcoreMesh(core_axis_name="c", subcore_axis_name="s"),
    scratch_types=[
        pltpu.VMEM_SHARED((_NP, _D), jnp.float32),
        pltpu.VMEM((_QC, _C), jnp.int32),
        pltpu.VMEM((_QC, _C), jnp.int32),
        pltpu.VMEM((_RC, _D), jnp.float32),
        pltpu.VMEM((_RC, _D), jnp.float32),
        pltpu.SemaphoreType.DMA,
        pltpu.SemaphoreType.DMA,
        pltpu.SemaphoreType.DMA,
        pltpu.SemaphoreType.DMA,
    ],
)


_BLK = 1024
_GRID = _NP // _BLK


def _mm_body(x_ref, w_ref, o_ref):
    o_ref[...] = jnp.dot(x_ref[...], w_ref[...],
                         preferred_element_type=jnp.float32)


_mm = pl.pallas_call(
    _mm_body,
    grid=(_GRID,),
    in_specs=[
        pl.BlockSpec((_BLK, _D), lambda i: (i, 0)),
        pl.BlockSpec((_D, _D), lambda i: (0, 0)),
    ],
    out_specs=pl.BlockSpec((_BLK, _D), lambda i: (i, 0)),
    out_shape=jax.ShapeDtypeStruct((_NP, _D), jnp.float32),
)


def _scale_body(xw_ref, ca_ref, cb_ref, z_ref, dinv_ref):
    deg = ca_ref[:, 0:1] + cb_ref[:, 0:1] + 1.0
    dinv = lax.rsqrt(deg)
    z_ref[...] = xw_ref[...] * dinv
    dinv_ref[...] = dinv


_scale = pl.pallas_call(
    _scale_body,
    grid=(_GRID,),
    in_specs=[
        pl.BlockSpec((_BLK, _D), lambda i: (i, 0)),
        pl.BlockSpec((_BLK, _D), lambda i: (i, 0)),
        pl.BlockSpec((_BLK, _D), lambda i: (i, 0)),
    ],
    out_specs=[
        pl.BlockSpec((_BLK, _D), lambda i: (i, 0)),
        pl.BlockSpec((_BLK, 1), lambda i: (i, 0)),
    ],
    out_shape=[
        jax.ShapeDtypeStruct((_NP, _D), jnp.float32),
        jax.ShapeDtypeStruct((_NP, 1), jnp.float32),
    ],
)


def _layer_body(sa_ref, sb_ref, z_ref, dinv_ref, b_ref, w_ref, zn_ref):
    dinv = dinv_ref[...]
    h = dinv * (sa_ref[...] + sb_ref[...] + z_ref[...]) + b_ref[...]
    h = jnp.maximum(h, 0.0)
    zn_ref[...] = jnp.dot(h, w_ref[...],
                          preferred_element_type=jnp.float32) * dinv


_layer = pl.pallas_call(
    _layer_body,
    grid=(_GRID,),
    in_specs=[
        pl.BlockSpec((_BLK, _D), lambda i: (i, 0)),
        pl.BlockSpec((_BLK, _D), lambda i: (i, 0)),
        pl.BlockSpec((_BLK, _D), lambda i: (i, 0)),
        pl.BlockSpec((_BLK, 1), lambda i: (i, 0)),
        pl.BlockSpec((1, _D), lambda i: (0, 0)),
        pl.BlockSpec((_D, _D), lambda i: (0, 0)),
    ],
    out_specs=pl.BlockSpec((_BLK, _D), lambda i: (i, 0)),
    out_shape=jax.ShapeDtypeStruct((_NP, _D), jnp.float32),
)


def _head_body(sa_ref, sb_ref, z_ref, dinv_ref, b_ref, batch_ref,
               wf1_ref, bf1_ref, wf2_ref, bf2_ref, o_ref):
    h = dinv_ref[...] * (sa_ref[...] + sb_ref[...] + z_ref[...]) + b_ref[...]
    gids = lax.broadcasted_iota(jnp.int32, (_G, _NP), 0)
    onehot = (batch_ref[...] == gids).astype(jnp.float32)
    sums = jnp.dot(onehot, h, preferred_element_type=jnp.float32)
    cnts = jnp.sum(onehot, axis=1, keepdims=True)
    g = sums / jnp.maximum(cnts, 1.0)
    a = jnp.maximum(
        jnp.dot(g, wf1_ref[...], preferred_element_type=jnp.float32)
        + bf1_ref[...], 0.0)
    o_ref[...] = (jnp.dot(a, wf2_ref[...], preferred_element_type=jnp.float32)
                  + bf2_ref[...])


_head = pl.pallas_call(
    _head_body,
    out_shape=jax.ShapeDtypeStruct((_G, 10), jnp.float32),
)


def kernel(x, edge_index, batch, W1, b1, W2, b2, W3, b3, Wf1, bf1, Wf2, bf2):
    src = edge_index[0].astype(jnp.int32).reshape(_E_ROWS, _C)
    dst = edge_index[1].astype(jnp.int32).reshape(_E_ROWS, _C)
    x_pad = jnp.pad(x, ((0, _NP - _N), (0, 0)))
    batch2d = jnp.pad(batch.astype(jnp.int32), (0, _NP - _N),
                      constant_values=_G).reshape(1, _NP)

    cnts = _sc_degree(dst)
    xw1 = _mm(x_pad, W1)
    z1, dinv = _scale(xw1, cnts[0], cnts[1])

    s1 = _sc_scatter(z1, src, dst)
    z2 = _layer(s1[0], s1[1], z1, dinv, b1.reshape(1, _D), W2)
    s2 = _sc_scatter(z2, src, dst)
    z3 = _layer(s2[0], s2[1], z2, dinv, b2.reshape(1, _D), W3)
    s3 = _sc_scatter(z3, src, dst)

    out = _head(s3[0], s3[1], z3, dinv, b3.reshape(1, _D), batch2d,
                Wf1, bf1.reshape(1, 64), Wf2, bf2.reshape(1, 10))
    return out


# speedup vs baseline: 23.6042x; 1.0065x over previous
import jax
import jax.numpy as jnp
from jax import lax
from jax.experimental import pallas as pl
from jax.experimental.pallas import tpu as pltpu
from jax.experimental.pallas import tpu_sc as plsc

_N = 10000
_NP = 10240
_E = 320000
_D = 128
_G = 64
_NC = 2
_NS = 16
_TILES = _NC * _NS
_C = 125
_E_ROWS = _E // _C
_CHUNKS = _E // (_TILES * _C)
_RPT = _NP // _NS
_RC = 128
_RCHUNKS = _RPT // _RC

_LANES = 16


def _zero_vmem(ref, nrows, ncols):
    zv = jnp.zeros((_LANES,), jnp.float32)

    def _row(i, _):
        def _col(k, _):
            ref[i, pl.ds(k * _LANES, _LANES)] = zv
            return 0
        return lax.fori_loop(0, ncols // _LANES, _col, 0)

    lax.fori_loop(0, nrows, _row, 0)


def _sc_degree_body(dst_hbm, out_hbm, accum, dstv, buf, sem):
    cid = lax.axis_index("c")
    sid = lax.axis_index("s")
    t = cid * _NS + sid

    pltpu.sync_copy(dst_hbm.at[pl.ds(t * _CHUNKS, _CHUNKS)], dstv)

    _zero_vmem(buf, _RC, _D)

    def _zacc(k, _):
        pltpu.sync_copy(buf, accum.at[pl.ds(sid * _RPT + k * _RC, _RC)])
        return 0
    lax.fori_loop(0, _RCHUNKS, _zacc, 0)

    onev = jnp.where(lax.iota(jnp.int32, _LANES) == 0, 1.0, 0.0)

    def _ones(i, _):
        buf[i, pl.ds(0, _LANES)] = onev
        return 0
    lax.fori_loop(0, _C, _ones, 0)

    plsc.subcore_barrier()

    descs = []
    for j in range(_CHUNKS):
        descs.append(pltpu.async_copy(buf.at[pl.ds(0, _C)],
                                      accum.at[dstv.at[j]], sem, add=True))
    for d in descs:
        d.wait()

    plsc.subcore_barrier()

    def _wout(k, _):
        r0 = sid * _RPT + k * _RC
        pltpu.sync_copy(accum.at[pl.ds(r0, _RC)],
                        out_hbm.at[cid].at[pl.ds(r0, _RC)])
        return 0
    lax.fori_loop(0, _RCHUNKS, _wout, 0)


_sc_degree = pl.kernel(
    _sc_degree_body,
    out_type=jax.ShapeDtypeStruct((_NC, _NP, _D), jnp.float32),
    mesh=plsc.VectorSubcoreMesh(core_axis_name="c", subcore_axis_name="s"),
    scratch_types=[
        pltpu.VMEM_SHARED((_NP, _D), jnp.float32),
        pltpu.VMEM((_CHUNKS, _C), jnp.int32),
        pltpu.VMEM((_RC, _D), jnp.float32),
        pltpu.SemaphoreType.DMA,
    ],
)


_QB = 5
_QC = _CHUNKS // _QB


def _sc_scatter_body(z_hbm, src_hbm, dst_hbm, out_hbm,
                     accum, srcv, dstv, bufa, bufb,
                     gsema, gsemb, ssema, ssemb):
    cid = lax.axis_index("c")
    sid = lax.axis_index("s")
    t = cid * _NS + sid

    _zero_vmem(bufa, _RC, _D)

    def _zacc(k, _):
        pltpu.sync_copy(bufa, accum.at[pl.ds(sid * _RPT + k * _RC, _RC)])
        return 0
    lax.fori_loop(0, _RCHUNKS, _zacc, 0)

    plsc.subcore_barrier()

    bufs = (bufa, bufb)
    gsems = (gsema, gsemb)
    ssems = (ssema, ssemb)

    def _qblock(q, _):
        base = t * _CHUNKS + q * _QC
        pltpu.sync_copy(src_hbm.at[pl.ds(base, _QC)], srcv)
        pltpu.sync_copy(dst_hbm.at[pl.ds(base, _QC)], dstv)

        gd = [None, None]
        sd = [None, None]
        gd[0] = pltpu.async_copy(z_hbm.at[srcv.at[0]],
                                 bufs[0].at[pl.ds(0, _C)], gsems[0])
        for j in range(_QC):
            p = j % 2
            o = (j + 1) % 2
            if j + 1 < _QC:
                if sd[o] is not None:
                    sd[o].wait()
                gd[o] = pltpu.async_copy(z_hbm.at[srcv.at[j + 1]],
                                         bufs[o].at[pl.ds(0, _C)], gsems[o])
            gd[p].wait()
            sd[p] = pltpu.async_copy(bufs[p].at[pl.ds(0, _C)],
                                     accum.at[dstv.at[j]], ssems[p], add=True)
        sd[0].wait()
        sd[1].wait()
        return 0
    lax.fori_loop(0, _QB, _qblock, 0)

    plsc.subcore_barrier()

    def _wout(k, _):
        r0 = sid * _RPT + k * _RC
        pltpu.sync_copy(accum.at[pl.ds(r0, _RC)],
                        out_hbm.at[cid].at[pl.ds(r0, _RC)])
        return 0
    lax.fori_loop(0, _RCHUNKS, _wout, 0)


_sc_scatter = pl.kernel(
    _sc_scatter_body,
    out_type=jax.ShapeDtypeStruct((_NC, _NP, _D), jnp.float32),
    mesh=plsc.VectorSubcoreMesh(core_axis_name="c", subcore_axis_name="s"),
    scratch_types=[
        pltpu.VMEM_SHARED((_NP, _D), jnp.float32),
        pltpu.VMEM((_QC, _C), jnp.int32),
        pltpu.VMEM((_QC, _C), jnp.int32),
        pltpu.VMEM((_RC, _D), jnp.float32),
        pltpu.VMEM((_RC, _D), jnp.float32),
        pltpu.SemaphoreType.DMA,
        pltpu.SemaphoreType.DMA,
        pltpu.SemaphoreType.DMA,
        pltpu.SemaphoreType.DMA,
    ],
)


_BLK = 1024
_GRID = _NP // _BLK


def _mmscale_body(x_ref, w_ref, ca_ref, cb_ref, z_ref, dinv_ref):
    deg = ca_ref[:, 0:1] + cb_ref[:, 0:1] + 1.0
    dinv = lax.rsqrt(deg)
    z_ref[...] = jnp.dot(x_ref[...], w_ref[...],
                         preferred_element_type=jnp.float32) * dinv
    dinv_ref[...] = dinv


_mmscale = pl.pallas_call(
    _mmscale_body,
    grid=(_GRID,),
    in_specs=[
        pl.BlockSpec((_BLK, _D), lambda i: (i, 0)),
        pl.BlockSpec((_D, _D), lambda i: (0, 0)),
        pl.BlockSpec((_BLK, _D), lambda i: (i, 0)),
        pl.BlockSpec((_BLK, _D), lambda i: (i, 0)),
    ],
    out_specs=[
        pl.BlockSpec((_BLK, _D), lambda i: (i, 0)),
        pl.BlockSpec((_BLK, 1), lambda i: (i, 0)),
    ],
    out_shape=[
        jax.ShapeDtypeStruct((_NP, _D), jnp.float32),
        jax.ShapeDtypeStruct((_NP, 1), jnp.float32),
    ],
)


def _layer_body(sa_ref, sb_ref, z_ref, dinv_ref, b_ref, w_ref, zn_ref):
    dinv = dinv_ref[...]
    h = dinv * (sa_ref[...] + sb_ref[...] + z_ref[...]) + b_ref[...]
    h = jnp.maximum(h, 0.0)
    zn_ref[...] = jnp.dot(h, w_ref[...],
                          preferred_element_type=jnp.float32) * dinv


_layer = pl.pallas_call(
    _layer_body,
    grid=(_GRID,),
    in_specs=[
        pl.BlockSpec((_BLK, _D), lambda i: (i, 0)),
        pl.BlockSpec((_BLK, _D), lambda i: (i, 0)),
        pl.BlockSpec((_BLK, _D), lambda i: (i, 0)),
        pl.BlockSpec((_BLK, 1), lambda i: (i, 0)),
        pl.BlockSpec((1, _D), lambda i: (0, 0)),
        pl.BlockSpec((_D, _D), lambda i: (0, 0)),
    ],
    out_specs=pl.BlockSpec((_BLK, _D), lambda i: (i, 0)),
    out_shape=jax.ShapeDtypeStruct((_NP, _D), jnp.float32),
)


def _head_body(sa_ref, sb_ref, z_ref, dinv_ref, b_ref, batch_ref,
               wf1_ref, bf1_ref, wf2_ref, bf2_ref, o_ref):
    h = dinv_ref[...] * (sa_ref[...] + sb_ref[...] + z_ref[...]) + b_ref[...]
    gids = lax.broadcasted_iota(jnp.int32, (_G, _NP), 0)
    onehot = (batch_ref[...] == gids).astype(jnp.float32)
    sums = jnp.dot(onehot, h, preferred_element_type=jnp.float32)
    cnts = jnp.sum(onehot, axis=1, keepdims=True)
    g = sums / jnp.maximum(cnts, 1.0)
    a = jnp.maximum(
        jnp.dot(g, wf1_ref[...], preferred_element_type=jnp.float32)
        + bf1_ref[...], 0.0)
    o_ref[...] = (jnp.dot(a, wf2_ref[...], preferred_element_type=jnp.float32)
                  + bf2_ref[...])


_head = pl.pallas_call(
    _head_body,
    out_shape=jax.ShapeDtypeStruct((_G, 10), jnp.float32),
)


def kernel(x, edge_index, batch, W1, b1, W2, b2, W3, b3, Wf1, bf1, Wf2, bf2):
    src = edge_index[0].astype(jnp.int32).reshape(_E_ROWS, _C)
    dst = edge_index[1].astype(jnp.int32).reshape(_E_ROWS, _C)
    x_pad = jnp.pad(x, ((0, _NP - _N), (0, 0)))
    batch2d = jnp.pad(batch.astype(jnp.int32), (0, _NP - _N),
                      constant_values=_G).reshape(1, _NP)

    cnts = _sc_degree(dst)
    z1, dinv = _mmscale(x_pad, W1, cnts[0], cnts[1])

    s1 = _sc_scatter(z1, src, dst)
    z2 = _layer(s1[0], s1[1], z1, dinv, b1.reshape(1, _D), W2)
    s2 = _sc_scatter(z2, src, dst)
    z3 = _layer(s2[0], s2[1], z2, dinv, b2.reshape(1, _D), W3)
    s3 = _sc_scatter(z3, src, dst)

    out = _head(s3[0], s3[1], z3, dinv, b3.reshape(1, _D), batch2d,
                Wf1, bf1.reshape(1, 64), Wf2, bf2.reshape(1, 10))
    return out


# speedup vs baseline: 25.0861x; 1.0628x over previous
import jax
import jax.numpy as jnp
from jax import lax
from jax.experimental import pallas as pl
from jax.experimental.pallas import tpu as pltpu
from jax.experimental.pallas import tpu_sc as plsc

_N = 10000
_NP = 10240
_E = 320000
_D = 128
_G = 64
_NC = 2
_NS = 16
_TILES = _NC * _NS
_C = 125
_E_ROWS = _E // _C
_CHUNKS = _E // (_TILES * _C)
_RPT = _NP // _NS
_RC = 128
_RCHUNKS = _RPT // _RC

_LANES = 16


def _zero_vmem(ref, nrows, ncols):
    zv = jnp.zeros((_LANES,), jnp.float32)

    def _row(i, _):
        def _col(k, _):
            ref[i, pl.ds(k * _LANES, _LANES)] = zv
            return 0
        return lax.fori_loop(0, ncols // _LANES, _col, 0)

    lax.fori_loop(0, nrows, _row, 0)


def _sc_degree_body(dst_hbm, out_hbm, accum, dstv, buf, sem):
    cid = lax.axis_index("c")
    sid = lax.axis_index("s")
    t = cid * _NS + sid

    pltpu.sync_copy(dst_hbm.at[pl.ds(t * _CHUNKS, _CHUNKS)], dstv)

    _zero_vmem(buf, _RC, _D)

    def _zacc(k, _):
        pltpu.sync_copy(buf, accum.at[pl.ds(sid * _RPT + k * _RC, _RC)])
        return 0
    lax.fori_loop(0, _RCHUNKS, _zacc, 0)

    onev = jnp.where(lax.iota(jnp.int32, _LANES) == 0, 1.0, 0.0)

    def _ones(i, _):
        buf[i, pl.ds(0, _LANES)] = onev
        return 0
    lax.fori_loop(0, _C, _ones, 0)

    plsc.subcore_barrier()

    descs = []
    for j in range(_CHUNKS):
        descs.append(pltpu.async_copy(buf.at[pl.ds(0, _C)],
                                      accum.at[dstv.at[j]], sem, add=True))
    for d in descs:
        d.wait()

    plsc.subcore_barrier()

    def _wout(k, _):
        r0 = sid * _RPT + k * _RC
        pltpu.sync_copy(accum.at[pl.ds(r0, _RC)],
                        out_hbm.at[cid].at[pl.ds(r0, _RC)])
        return 0
    lax.fori_loop(0, _RCHUNKS, _wout, 0)


_sc_degree = pl.kernel(
    _sc_degree_body,
    out_type=jax.ShapeDtypeStruct((_NC, _NP, _D), jnp.float32),
    mesh=plsc.VectorSubcoreMesh(core_axis_name="c", subcore_axis_name="s"),
    scratch_types=[
        pltpu.VMEM_SHARED((_NP, _D), jnp.float32),
        pltpu.VMEM((_CHUNKS, _C), jnp.int32),
        pltpu.VMEM((_RC, _D), jnp.float32),
        pltpu.SemaphoreType.DMA,
    ],
)


_QB = 5
_QC = _CHUNKS // _QB


def _sc_scatter_body(z_hbm, src_hbm, dst_hbm, out_hbm,
                     accum, srcv, dstv, bufa, bufb,
                     gsema, gsemb, ssema, ssemb, isem):
    cid = lax.axis_index("c")
    sid = lax.axis_index("s")
    t = cid * _NS + sid

    _zero_vmem(bufa, _RC, _D)

    def _zacc(k, _):
        pltpu.sync_copy(bufa, accum.at[pl.ds(sid * _RPT + k * _RC, _RC)])
        return 0
    lax.fori_loop(0, _RCHUNKS, _zacc, 0)

    plsc.subcore_barrier()

    bufs = (bufa, bufb)
    gsems = (gsema, gsemb)
    ssems = (ssema, ssemb)

    pltpu.sync_copy(src_hbm.at[pl.ds(t * _CHUNKS, _QC)], srcv.at[0])
    pltpu.sync_copy(dst_hbm.at[pl.ds(t * _CHUNKS, _QC)], dstv.at[0])
    gd = [None, None]
    sd = [None, None]
    idescs = None
    gd[0] = pltpu.async_copy(z_hbm.at[srcv.at[0, 0]],
                             bufs[0].at[pl.ds(0, _C)], gsems[0])
    for j in range(_CHUNKS):
        q, r = divmod(j, _QC)
        b = q % 2
        p = j % 2
        o = (j + 1) % 2
        if r == 1 and q + 1 < _QB:
            nb = (q + 1) % 2
            base = t * _CHUNKS + (q + 1) * _QC
            idescs = (
                pltpu.async_copy(src_hbm.at[pl.ds(base, _QC)],
                                 srcv.at[nb], isem),
                pltpu.async_copy(dst_hbm.at[pl.ds(base, _QC)],
                                 dstv.at[nb], isem),
            )
        if j + 1 < _CHUNKS:
            nq, nr = divmod(j + 1, _QC)
            if nr == 0:
                idescs[0].wait()
                idescs[1].wait()
            if sd[o] is not None:
                sd[o].wait()
            gd[o] = pltpu.async_copy(z_hbm.at[srcv.at[nq % 2, nr]],
                                     bufs[o].at[pl.ds(0, _C)], gsems[o])
        gd[p].wait()
        sd[p] = pltpu.async_copy(bufs[p].at[pl.ds(0, _C)],
                                 accum.at[dstv.at[b, r]], ssems[p], add=True)
    sd[0].wait()
    sd[1].wait()

    plsc.subcore_barrier()

    def _wout(k, _):
        r0 = sid * _RPT + k * _RC
        pltpu.sync_copy(accum.at[pl.ds(r0, _RC)],
                        out_hbm.at[cid].at[pl.ds(r0, _RC)])
        return 0
    lax.fori_loop(0, _RCHUNKS, _wout, 0)


_sc_scatter = pl.kernel(
    _sc_scatter_body,
    out_type=jax.ShapeDtypeStruct((_NC, _NP, _D), jnp.float32),
    mesh=plsc.VectorSubcoreMesh(core_axis_name="c", subcore_axis_name="s"),
    scratch_types=[
        pltpu.VMEM_SHARED((_NP, _D), jnp.float32),
        pltpu.VMEM((2, _QC, _C), jnp.int32),
        pltpu.VMEM((2, _QC, _C), jnp.int32),
        pltpu.VMEM((_RC, _D), jnp.float32),
        pltpu.VMEM((_RC, _D), jnp.float32),
        pltpu.SemaphoreType.DMA,
        pltpu.SemaphoreType.DMA,
        pltpu.SemaphoreType.DMA,
        pltpu.SemaphoreType.DMA,
        pltpu.SemaphoreType.DMA,
    ],
)


_BLK = 1024
_GRID = _NP // _BLK


def _mmscale_body(x_ref, w_ref, ca_ref, cb_ref, z_ref, dinv_ref):
    deg = ca_ref[:, 0:1] + cb_ref[:, 0:1] + 1.0
    dinv = lax.rsqrt(deg)
    z_ref[...] = jnp.dot(x_ref[...], w_ref[...],
                         preferred_element_type=jnp.float32) * dinv
    dinv_ref[...] = dinv


_mmscale = pl.pallas_call(
    _mmscale_body,
    grid=(_GRID,),
    in_specs=[
        pl.BlockSpec((_BLK, _D), lambda i: (i, 0)),
        pl.BlockSpec((_D, _D), lambda i: (0, 0)),
        pl.BlockSpec((_BLK, _D), lambda i: (i, 0)),
        pl.BlockSpec((_BLK, _D), lambda i: (i, 0)),
    ],
    out_specs=[
        pl.BlockSpec((_BLK, _D), lambda i: (i, 0)),
        pl.BlockSpec((_BLK, 1), lambda i: (i, 0)),
    ],
    out_shape=[
        jax.ShapeDtypeStruct((_NP, _D), jnp.float32),
        jax.ShapeDtypeStruct((_NP, 1), jnp.float32),
    ],
)


def _layer_body(sa_ref, sb_ref, z_ref, dinv_ref, b_ref, w_ref, zn_ref):
    dinv = dinv_ref[...]
    h = dinv * (sa_ref[...] + sb_ref[...] + z_ref[...]) + b_ref[...]
    h = jnp.maximum(h, 0.0)
    zn_ref[...] = jnp.dot(h, w_ref[...],
                          preferred_element_type=jnp.float32) * dinv


_layer = pl.pallas_call(
    _layer_body,
    grid=(_GRID,),
    in_specs=[
        pl.BlockSpec((_BLK, _D), lambda i: (i, 0)),
        pl.BlockSpec((_BLK, _D), lambda i: (i, 0)),
        pl.BlockSpec((_BLK, _D), lambda i: (i, 0)),
        pl.BlockSpec((_BLK, 1), lambda i: (i, 0)),
        pl.BlockSpec((1, _D), lambda i: (0, 0)),
        pl.BlockSpec((_D, _D), lambda i: (0, 0)),
    ],
    out_specs=pl.BlockSpec((_BLK, _D), lambda i: (i, 0)),
    out_shape=jax.ShapeDtypeStruct((_NP, _D), jnp.float32),
)


def _head_body(sa_ref, sb_ref, z_ref, dinv_ref, b_ref, batch_ref,
               wf1_ref, bf1_ref, wf2_ref, bf2_ref, o_ref):
    h = dinv_ref[...] * (sa_ref[...] + sb_ref[...] + z_ref[...]) + b_ref[...]
    gids = lax.broadcasted_iota(jnp.int32, (_G, _NP), 0)
    onehot = (batch_ref[...] == gids).astype(jnp.float32)
    sums = jnp.dot(onehot, h, preferred_element_type=jnp.float32)
    cnts = jnp.sum(onehot, axis=1, keepdims=True)
    g = sums / jnp.maximum(cnts, 1.0)
    a = jnp.maximum(
        jnp.dot(g, wf1_ref[...], preferred_element_type=jnp.float32)
        + bf1_ref[...], 0.0)
    o_ref[...] = (jnp.dot(a, wf2_ref[...], preferred_element_type=jnp.float32)
                  + bf2_ref[...])


_head = pl.pallas_call(
    _head_body,
    out_shape=jax.ShapeDtypeStruct((_G, 10), jnp.float32),
)


def kernel(x, edge_index, batch, W1, b1, W2, b2, W3, b3, Wf1, bf1, Wf2, bf2):
    src = edge_index[0].astype(jnp.int32).reshape(_E_ROWS, _C)
    dst = edge_index[1].astype(jnp.int32).reshape(_E_ROWS, _C)
    x_pad = jnp.pad(x, ((0, _NP - _N), (0, 0)))
    batch2d = jnp.pad(batch.astype(jnp.int32), (0, _NP - _N),
                      constant_values=_G).reshape(1, _NP)

    cnts = _sc_degree(dst)
    z1, dinv = _mmscale(x_pad, W1, cnts[0], cnts[1])

    s1 = _sc_scatter(z1, src, dst)
    z2 = _layer(s1[0], s1[1], z1, dinv, b1.reshape(1, _D), W2)
    s2 = _sc_scatter(z2, src, dst)
    z3 = _layer(s2[0], s2[1], z2, dinv, b2.reshape(1, _D), W3)
    s3 = _sc_scatter(z3, src, dst)

    out = _head(s3[0], s3[1], z3, dinv, b3.reshape(1, _D), batch2d,
                Wf1, bf1.reshape(1, 64), Wf2, bf2.reshape(1, 10))
    return out
